# baseline (speedup 1.0000x reference)
import functools

import jax
import jax.numpy as jnp
from jax import lax
from jax.experimental import pallas as pl
from jax.experimental.pallas import tpu as pltpu
from jax.experimental.pallas import tpu_sc as plsc

NC = 2
NS = 16
NW = NC * NS
L = 16


def _sigmoid(x):
    e = jnp.exp(-jnp.abs(x))
    return jnp.where(x >= 0, 1.0 / (1.0 + e), e / (1.0 + e))


def _tanh(x):
    e = jnp.exp(-2.0 * jnp.abs(x))
    t = (1.0 - e) / (1.0 + e)
    return jnp.where(x >= 0, t, -t)


@jax.jit
def kernel(inputs, state, gate_kernel_users, gate_kernel_items, gate_bias,
           candidate_kernel_users, candidate_kernel_items, candidate_bias):
    B, H = state.shape
    BPW = B // NW
    EPW = BPW * H
    u_idx = inputs[:, 0].astype(jnp.int32)
    i_idx = inputs[:, 1].astype(jnp.int32)
    U1 = gate_kernel_users.shape[0]
    GR = 2 * H * H // 128
    CR = H * H // 128
    gku4 = gate_kernel_users.reshape(U1 * GR, 128)
    cku4 = candidate_kernel_users.reshape(U1 * CR, 128)
    NI = 100000
    gki4 = gate_kernel_items[:NI].reshape(NI * 2 * H // 128, 128)
    cki4 = candidate_kernel_items[:NI].reshape(NI * H // 128, 128)
    ar4 = jnp.arange(GR, dtype=jnp.int32)
    ar2 = jnp.arange(CR, dtype=jnp.int32)
    gur = (u_idx[:, None] * GR + ar4[None, :]).reshape(B * GR)
    cur = (u_idx[:, None] * CR + ar2[None, :]).reshape(B * CR)
    gr = i_idx // 4
    gc = (i_idx % 4) * (2 * H) + H
    cr = i_idx // 8
    cc = (i_idx % 8) * H
    s1 = state.reshape(B * H)
    GUR = BPW * GR
    CUR = BPW * CR
    WAVE = BPW // 2

    mesh = plsc.VectorSubcoreMesh(
        core_axis_name="c", subcore_axis_name="s",
        num_cores=NC, num_subcores=NS)

    @functools.partial(
        pl.kernel,
        out_type=jax.ShapeDtypeStruct((B * H,), jnp.float32),
        mesh=mesh,
        scratch_types=[
            pltpu.VMEM((GUR,), jnp.int32),
            pltpu.VMEM((CUR,), jnp.int32),
            pltpu.VMEM((BPW,), jnp.int32),
            pltpu.VMEM((BPW,), jnp.int32),
            pltpu.VMEM((BPW,), jnp.int32),
            pltpu.VMEM((BPW,), jnp.int32),
            pltpu.VMEM((EPW,), jnp.float32),
            pltpu.VMEM((GUR, 128), jnp.float32),
            pltpu.VMEM((CUR, 128), jnp.float32),
            pltpu.VMEM((WAVE, 128), jnp.float32),
            pltpu.VMEM((WAVE, 128), jnp.float32),
            pltpu.VMEM((EPW,), jnp.float32),
            pltpu.VMEM((EPW,), jnp.float32),
            pltpu.VMEM((2 * H,), jnp.float32),
            pltpu.VMEM((H,), jnp.float32),
            pltpu.VMEM((EPW,), jnp.float32),
            pltpu.SemaphoreType.DMA,
            pltpu.SemaphoreType.DMA,
            pltpu.SemaphoreType.DMA,
            pltpu.SemaphoreType.DMA,
        ],
    )
    def run(gur_hbm, cur_hbm, gr_hbm, cr_hbm, gc_hbm, cc_hbm, s_hbm,
            gku_hbm, cku_hbm, gki_hbm, cki_hbm, gb_hbm, cb_hbm, out_hbm,
            gur_v, cur_v, gr_v, cr_v, gc_v, cc_v, s_v, wg_v, wc_v,
            giw_v, ciw_v, gi_v, ci_v, gb_v, cb_v, o_v,
            sem0, sem1, sem2, sem3):
        wid = lax.axis_index("s") * NC + lax.axis_index("c")
        base = wid * BPW
        pltpu.sync_copy(gur_hbm.at[pl.ds(wid * GUR, GUR)], gur_v)
        pltpu.sync_copy(cur_hbm.at[pl.ds(wid * CUR, CUR)], cur_v)
        cp0 = pltpu.async_copy(gku_hbm.at[gur_v], wg_v, sem0)
        cp1 = pltpu.async_copy(cku_hbm.at[cur_v], wc_v, sem1)
        pltpu.sync_copy(gr_hbm.at[pl.ds(base, BPW)], gr_v)
        pltpu.sync_copy(cr_hbm.at[pl.ds(base, BPW)], cr_v)
        pltpu.sync_copy(gc_hbm.at[pl.ds(base, BPW)], gc_v)
        pltpu.sync_copy(cc_hbm.at[pl.ds(base, BPW)], cc_v)
        pltpu.sync_copy(s_hbm.at[pl.ds(base * H, EPW)], s_v)
        pltpu.sync_copy(gb_hbm, gb_v)
        pltpu.sync_copy(cb_hbm, cb_v)

        for w in range(2):
            gcp = pltpu.async_copy(
                gki_hbm.at[gr_v.at[pl.ds(w * WAVE, WAVE)]], giw_v, sem2)
            ccp = pltpu.async_copy(
                cki_hbm.at[cr_v.at[pl.ds(w * WAVE, WAVE)]], ciw_v, sem3)
            gcp.wait()
            ccp.wait()
            for g in range(WAVE // L):
                gcv = gc_v[pl.ds(w * WAVE + g * L, L)]
                ccv = cc_v[pl.ds(w * WAVE + g * L, L)]
                for l in range(L):
                    b = w * WAVE + g * L + l
                    r = g * L + l
                    gi_v[pl.ds(b * H, H)] = giw_v[r, pl.ds(gcv[l], H)]
                    ci_v[pl.ds(b * H, H)] = ciw_v[r, pl.ds(ccv[l], H)]

        cp0.wait()
        cp1.wait()

        gbias_hi = gb_v[pl.ds(H, H)]
        cbias = cb_v[...]

        def elem(b, carry):
            acc_u = gbias_hi + gi_v[pl.ds(b * H, H)]
            acc_c = cbias + ci_v[pl.ds(b * H, H)]
            sb = s_v[pl.ds(b * H, H)]
            for h in range(H):
                sh = sb[h]
                acc_u = acc_u + sh * wg_v[GR * b + (h * 2 * H + H) // 128,
                                          pl.ds((h * 2 * H + H) % 128, H)]
                acc_c = acc_c + sh * wc_v[CR * b + (h * H) // 128,
                                          pl.ds((h * H) % 128, H)]
            u_gate = _sigmoid(acc_u)
            c = _tanh(acc_c)
            o_v[pl.ds(b * H, H)] = u_gate * sb + (1.0 - u_gate) * c
            return carry

        lax.fori_loop(0, BPW, elem, 0)
        pltpu.sync_copy(o_v, out_hbm.at[pl.ds(base * H, EPW)])

    out = run(gur, cur, gr, cr, gc, cc, s1, gku4, cku4, gki4, cki4,
              gate_bias, candidate_bias)
    return out.reshape(B, H)


# speedup vs baseline: 3.9925x; 3.9925x over previous
import functools

import jax
import jax.numpy as jnp
from jax import lax
from jax.experimental import pallas as pl
from jax.experimental.pallas import tpu as pltpu
from jax.experimental.pallas import tpu_sc as plsc

NC = 2
NS = 16
NW = NC * NS


def _sigmoid(x):
    e = jnp.exp(-jnp.abs(x))
    return jnp.where(x >= 0, 1.0 / (1.0 + e), e / (1.0 + e))


def _tanh(x):
    e = jnp.exp(-2.0 * jnp.abs(x))
    t = (1.0 - e) / (1.0 + e)
    return jnp.where(x >= 0, t, -t)


@jax.jit
def kernel(inputs, state, gate_kernel_users, gate_kernel_items, gate_bias,
           candidate_kernel_users, candidate_kernel_items, candidate_bias):
    B, H = state.shape
    BPW = B // NW
    EPW = BPW * H
    u_idx = inputs[:, 0].astype(jnp.int32)
    i_idx = inputs[:, 1].astype(jnp.int32)
    U1 = gate_kernel_users.shape[0]
    wgu = gate_kernel_users[:, :, H:].reshape(U1, H * H)
    wcu = candidate_kernel_users.reshape(U1, H * H)
    ar = jnp.arange(H, dtype=jnp.int32)
    gidx = (i_idx[:, None] * (2 * H) + (H + ar)[None, :]).reshape(B * H // 128, 128)
    cidx = (i_idx[:, None] * H + ar[None, :]).reshape(B * H // 128, 128)
    gkif = gate_kernel_items.reshape(gate_kernel_items.size)
    ckif = candidate_kernel_items.reshape(candidate_kernel_items.size)
    s1 = state.reshape(B * H)
    NCHUNK = EPW // 128

    mesh = plsc.VectorSubcoreMesh(
        core_axis_name="c", subcore_axis_name="s",
        num_cores=NC, num_subcores=NS)

    @functools.partial(
        pl.kernel,
        out_type=jax.ShapeDtypeStruct((B * H,), jnp.float32),
        mesh=mesh,
        scratch_types=[
            pltpu.VMEM((BPW,), jnp.int32),
            pltpu.VMEM((NCHUNK, 128), jnp.int32),
            pltpu.VMEM((NCHUNK, 128), jnp.int32),
            pltpu.VMEM((EPW,), jnp.float32),
            pltpu.VMEM((BPW, H * H), jnp.float32),
            pltpu.VMEM((NCHUNK, 128), jnp.float32),
            pltpu.VMEM((BPW, H * H), jnp.float32),
            pltpu.VMEM((NCHUNK, 128), jnp.float32),
            pltpu.VMEM((2 * H,), jnp.float32),
            pltpu.VMEM((H,), jnp.float32),
            pltpu.VMEM((EPW,), jnp.float32),
            pltpu.SemaphoreType.DMA,
            pltpu.SemaphoreType.DMA,
            pltpu.SemaphoreType.DMA,
            pltpu.SemaphoreType.DMA,
        ],
    )
    def run(u_hbm, gidx_hbm, cidx_hbm, s_hbm, gku_hbm, gkif_hbm, cku_hbm,
            ckif_hbm, gb_hbm, cb_hbm, out_hbm,
            u_v, gx_v, cx_v, s_v, wg_v, gi_v, wc_v, ci_v, gb_v, cb_v, o_v,
            sem0, sem1, sem2, sem3):
        wid = lax.axis_index("s") * NC + lax.axis_index("c")
        base = wid * BPW
        pltpu.sync_copy(u_hbm.at[pl.ds(base, BPW)], u_v)
        cp0 = pltpu.async_copy(gku_hbm.at[u_v], wg_v, sem0)
        cp1 = pltpu.async_copy(cku_hbm.at[u_v], wc_v, sem1)
        pltpu.sync_copy(gidx_hbm.at[pl.ds(wid * NCHUNK, NCHUNK)], gx_v)
        pltpu.sync_copy(cidx_hbm.at[pl.ds(wid * NCHUNK, NCHUNK)], cx_v)
        item_cps = []
        for j in range(NCHUNK):
            item_cps.append(
                pltpu.async_copy(gkif_hbm.at[gx_v.at[j]], gi_v.at[j], sem2))
            item_cps.append(
                pltpu.async_copy(ckif_hbm.at[cx_v.at[j]], ci_v.at[j], sem3))
        pltpu.sync_copy(s_hbm.at[pl.ds(base * H, EPW)], s_v)
        pltpu.sync_copy(gb_hbm, gb_v)
        pltpu.sync_copy(cb_hbm, cb_v)
        for cp in item_cps:
            cp.wait()
        cp0.wait()
        cp1.wait()

        gbias_hi = gb_v[pl.ds(H, H)]
        cbias = cb_v[...]

        def elem(b, carry):
            j = b // 8
            col = (b % 8) * H
            acc_u = gbias_hi + gi_v[j, pl.ds(col, H)]
            acc_c = cbias + ci_v[j, pl.ds(col, H)]
            sb = s_v[pl.ds(b * H, H)]
            for h in range(H):
                sh = sb[h]
                acc_u = acc_u + sh * wg_v[b, pl.ds(h * H, H)]
                acc_c = acc_c + sh * wc_v[b, pl.ds(h * H, H)]
            u_gate = _sigmoid(acc_u)
            c = _tanh(acc_c)
            o_v[pl.ds(b * H, H)] = u_gate * sb + (1.0 - u_gate) * c
            return carry

        lax.fori_loop(0, BPW, elem, 0)
        pltpu.sync_copy(o_v, out_hbm.at[pl.ds(base * H, EPW)])

    out = run(u_idx, gidx, cidx, s1, wgu, gkif, wcu, ckif,
              gate_bias, candidate_bias)
    return out.reshape(B, H)


# speedup vs baseline: 7.2032x; 1.8042x over previous
import functools

import jax
import jax.numpy as jnp
from jax import lax
from jax.experimental import pallas as pl
from jax.experimental.pallas import tpu as pltpu
from jax.experimental.pallas import tpu_sc as plsc

NC = 2
NS = 16
NW = NC * NS
L = 16


def _sigmoid(x):
    e = jnp.exp(-jnp.abs(x))
    return jnp.where(x >= 0, 1.0 / (1.0 + e), e / (1.0 + e))


def _tanh(x):
    e = jnp.exp(-2.0 * jnp.abs(x))
    t = (1.0 - e) / (1.0 + e)
    return jnp.where(x >= 0, t, -t)


@jax.jit
def kernel(inputs, state, gate_kernel_users, gate_kernel_items, gate_bias,
           candidate_kernel_users, candidate_kernel_items, candidate_bias):
    B, H = state.shape
    BPW = B // NW
    NGALL = B // L
    u_idx = inputs[:, 0].astype(jnp.int32)
    i_idx = inputs[:, 1].astype(jnp.int32)
    U1 = gate_kernel_users.shape[0]
    GT = gate_kernel_users.transpose(1, 2, 0).reshape(2 * H * H, U1)
    CT = candidate_kernel_users.transpose(1, 2, 0).reshape(H * H, U1)
    GIT = gate_kernel_items.T
    CIT = candidate_kernel_items.T
    ST = state.T
    W0 = (U1 // 2) // 128 * 128
    W1 = U1 - W0
    NF = 2 * H * H + 2 * H
    RPT = NF // NW

    mesh = plsc.VectorSubcoreMesh(
        core_axis_name="c", subcore_axis_name="s",
        num_cores=NC, num_subcores=NS)

    @functools.partial(
        pl.kernel,
        out_type=jax.ShapeDtypeStruct((NF, B), jnp.float32),
        mesh=mesh,
        scratch_types=[
            pltpu.VMEM((B,), jnp.int32),
            pltpu.VMEM((B,), jnp.int32),
            pltpu.VMEM((1, W0), jnp.float32),
            pltpu.VMEM((1, W1), jnp.float32),
            pltpu.VMEM((1, B), jnp.float32),
            pltpu.SemaphoreType.DMA,
            pltpu.SemaphoreType.DMA,
        ],
        compiler_params=pltpu.CompilerParams(needs_layout_passes=False),
    )
    def gather_rows(u_hbm, i_hbm, gt_hbm, ct_hbm, git_hbm, cit_hbm, x_hbm,
                    uv, iv, bufa, bufb, orow, sema, semb):
        wid = lax.axis_index("s") * NC + lax.axis_index("c")
        pltpu.sync_copy(u_hbm, uv)
        pltpu.sync_copy(i_hbm, iv)
        zero16 = jnp.zeros((L,), jnp.int32)
        iota16 = lax.iota(jnp.int32, L)

        def do_row(src_hbm, src_row, ids_v, out_row):
            cpa = pltpu.async_copy(
                src_hbm.at[pl.ds(src_row, 1), pl.ds(0, W0)], bufa, sema)
            cpb = pltpu.async_copy(
                src_hbm.at[pl.ds(src_row, 1), pl.ds(W0, W1)], bufb, semb)
            cpa.wait()

            def ga(g, carry):
                u = ids_v[pl.ds(g * L, L)]
                ma = u < W0
                va = plsc.load_gather(bufa, [zero16, u], mask=ma)
                orow[0, pl.ds(g * L, L)] = va
                return carry

            lax.fori_loop(0, NGALL, ga, 0)
            cpb.wait()

            def gb(g, carry):
                u = ids_v[pl.ds(g * L, L)]
                mb = u >= W0
                vb = plsc.load_gather(bufb, [zero16, u - W0], mask=mb)
                plsc.store_scatter(orow, [zero16, g * L + iota16], vb,
                                   mask=mb)
                return carry

            lax.fori_loop(0, NGALL, gb, 0)
            pltpu.sync_copy(orow, x_hbm.at[pl.ds(out_row, 1), :])

        for j in range(8):
            r = wid * 8 + j
            h = r // H
            k = lax.rem(r, H)
            do_row(gt_hbm, h * 2 * H + H + k, uv, r)
        for j in range(8):
            r = wid * 8 + j
            do_row(ct_hbm, r, uv, 2 * H * H // 2 + r)
        @pl.when(wid < NS)
        def _():
            do_row(git_hbm, H + wid, iv, 2 * H * H + wid)

        @pl.when(wid >= NS)
        def _():
            do_row(cit_hbm, wid - NS, iv, 2 * H * H + H + wid - NS)

    @functools.partial(
        pl.kernel,
        out_type=jax.ShapeDtypeStruct((H, B), jnp.float32),
        mesh=mesh,
        scratch_types=[
            pltpu.VMEM((NF, BPW), jnp.float32),
            pltpu.VMEM((H, BPW), jnp.float32),
            pltpu.VMEM((2 * H,), jnp.float32),
            pltpu.VMEM((H,), jnp.float32),
            pltpu.VMEM((H, BPW), jnp.float32),
        ],
    )
    def compute(x_hbm, st_hbm, gb_hbm, cb_hbm, out_hbm,
                xv, sv, gbv, cbv, ov):
        wid = lax.axis_index("s") * NC + lax.axis_index("c")
        base = wid * BPW
        pltpu.sync_copy(x_hbm.at[:, pl.ds(base, BPW)], xv)
        pltpu.sync_copy(st_hbm.at[:, pl.ds(base, BPW)], sv)
        pltpu.sync_copy(gb_hbm, gbv)
        pltpu.sync_copy(cb_hbm, cbv)
        gbh = gbv[pl.ds(H, H)]
        cbh = cbv[...]

        def group(g, carry):
            gs = pl.ds(g * L, L)
            sh = [sv[h, gs] for h in range(H)]
            for k in range(H):
                acc_u = xv[2 * H * H + k, gs] + gbh[k]
                acc_c = xv[2 * H * H + H + k, gs] + cbh[k]
                for h in range(H):
                    acc_u = acc_u + sh[h] * xv[h * H + k, gs]
                    acc_c = acc_c + sh[h] * xv[H * H + h * H + k, gs]
                u_gate = _sigmoid(acc_u)
                c = _tanh(acc_c)
                ov[k, gs] = u_gate * sh[k] + (1.0 - u_gate) * c
            return carry

        lax.fori_loop(0, BPW // L, group, 0)
        pltpu.sync_copy(ov, out_hbm.at[:, pl.ds(base, BPW)])

    x = gather_rows(u_idx, i_idx, GT, CT, GIT, CIT)
    out = compute(x, ST, gate_bias, candidate_bias)
    return out.T


# speedup vs baseline: 8.2197x; 1.1411x over previous
import functools

import jax
import jax.numpy as jnp
from jax import lax
from jax.experimental import pallas as pl
from jax.experimental.pallas import tpu as pltpu
from jax.experimental.pallas import tpu_sc as plsc

NC = 2
NS = 16
NW = NC * NS
L = 16


def _sigmoid(x):
    e = jnp.exp(-jnp.abs(x))
    return jnp.where(x >= 0, 1.0 / (1.0 + e), e / (1.0 + e))


def _tanh(x):
    e = jnp.exp(-2.0 * jnp.abs(x))
    t = (1.0 - e) / (1.0 + e)
    return jnp.where(x >= 0, t, -t)


@jax.jit
def kernel(inputs, state, gate_kernel_users, gate_kernel_items, gate_bias,
           candidate_kernel_users, candidate_kernel_items, candidate_bias):
    B, H = state.shape
    BPW = B // NW
    NGALL = B // L
    u_idx = inputs[:, 0].astype(jnp.int32)
    i_idx = inputs[:, 1].astype(jnp.int32)
    U1 = gate_kernel_users.shape[0]
    GT = gate_kernel_users.transpose(1, 2, 0).reshape(2 * H * H, U1)
    CT = candidate_kernel_users.transpose(1, 2, 0).reshape(H * H, U1)
    GIT = gate_kernel_items.T
    CIT = candidate_kernel_items.T
    ST = state.T
    T0 = (U1 // 3) // 128 * 128
    OFFS = (0, T0, 2 * T0)
    TW = (T0, T0, U1 - 2 * T0)
    TMAX = max(TW)
    NF = 2 * H * H + 2 * H
    NRING = 16

    mesh = plsc.VectorSubcoreMesh(
        core_axis_name="c", subcore_axis_name="s",
        num_cores=NC, num_subcores=NS)

    @functools.partial(
        pl.kernel,
        out_type=jax.ShapeDtypeStruct((NF, B), jnp.float32),
        mesh=mesh,
        scratch_types=[
            pltpu.VMEM((B,), jnp.int32),
            pltpu.VMEM((B,), jnp.int32),
            pltpu.VMEM((1, TMAX), jnp.float32),
            pltpu.VMEM((1, TMAX), jnp.float32),
            pltpu.VMEM((1, TMAX), jnp.float32),
            pltpu.VMEM((1, B), jnp.float32),
            pltpu.VMEM((1, B), jnp.float32),
            pltpu.SemaphoreType.DMA,
            pltpu.SemaphoreType.DMA,
            pltpu.SemaphoreType.DMA,
            pltpu.SemaphoreType.DMA,
        ],
        compiler_params=pltpu.CompilerParams(needs_layout_passes=False),
    )
    def gather_rows(u_hbm, i_hbm, gt_hbm, ct_hbm, git_hbm, cit_hbm, x_hbm,
                    uv, iv, b0, b1, b2, oe, oo, s0, s1, s2, so):
        wid = lax.axis_index("s") * NC + lax.axis_index("c")
        pltpu.sync_copy(u_hbm, uv)
        pltpu.sync_copy(i_hbm, iv)
        zero16 = jnp.zeros((L,), jnp.int32)
        iota16 = lax.iota(jnp.int32, L)
        bufs = (b0, b1, b2)
        sems = (s0, s1, s2)
        orows = (oe, oo)

        def seg_copy(src_hbm, src_row, t, bi):
            return pltpu.make_async_copy(
                src_hbm.at[pl.ds(src_row, 1), pl.ds(OFFS[t], TW[t])],
                bufs[bi].at[:, pl.ds(0, TW[t])], sems[bi])

        def seg_gather(ids_v, t, bi, orow):
            lo, hi = OFFS[t], OFFS[t] + TW[t]

            def body(g, carry):
                u = ids_v[pl.ds(g * L, L)]
                m = (u >= lo) & (u < hi)
                v = plsc.load_gather(bufs[bi], [zero16, u - lo], mask=m)
                if t == 0:
                    orow[0, pl.ds(g * L, L)] = v
                else:
                    plsc.store_scatter(orow, [zero16, g * L + iota16], v,
                                       mask=m)
                return carry

            lax.fori_loop(0, NGALL, body, 0)

        def out_copy(orow, out_row):
            return pltpu.make_async_copy(
                orow, x_hbm.at[pl.ds(out_row, 1), :], so)

        rows = []
        for j in range(8):
            r = wid * 8 + j
            h = r // H
            k = lax.rem(r, H)
            rows.append((gt_hbm, h * 2 * H + H + k, r))
        for j in range(8):
            r = wid * 8 + j
            rows.append((ct_hbm, r, H * H + r))

        segs = [(ri, t) for ri in range(NRING) for t in range(3)]
        for i in range(3):
            ri, t = segs[i]
            seg_copy(rows[ri][0], rows[ri][1], t, i).start()
        for i, (ri, t) in enumerate(segs):
            bi = i % 3
            src_hbm, src_row, out_row = rows[ri]
            seg_copy(src_hbm, src_row, t, bi).wait()
            if t == 0 and ri >= 2:
                out_copy(orows[ri % 2], rows[ri - 2][2]).wait()
            seg_gather(uv, t, bi, orows[ri % 2])
            if i + 3 < len(segs):
                nri, nt = segs[i + 3]
                seg_copy(rows[nri][0], rows[nri][1], nt, bi).start()
            if t == 2:
                out_copy(orows[ri % 2], out_row).start()
        out_copy(orows[0], rows[NRING - 2][2]).wait()
        out_copy(orows[1], rows[NRING - 1][2]).wait()

        @pl.when(wid < NS)
        def _():
            for t in range(3):
                seg_copy(git_hbm, H + wid, t, t).start()
            for t in range(3):
                seg_copy(git_hbm, H + wid, t, t).wait()
                seg_gather(iv, t, t, oe)
            pltpu.sync_copy(oe, x_hbm.at[pl.ds(2 * H * H + wid, 1), :])

        @pl.when(wid >= NS)
        def _():
            for t in range(3):
                seg_copy(cit_hbm, wid - NS, t, t).start()
            for t in range(3):
                seg_copy(cit_hbm, wid - NS, t, t).wait()
                seg_gather(iv, t, t, oe)
            pltpu.sync_copy(
                oe, x_hbm.at[pl.ds(2 * H * H + H + wid - NS, 1), :])

    @functools.partial(
        pl.kernel,
        out_type=jax.ShapeDtypeStruct((H, B), jnp.float32),
        mesh=mesh,
        scratch_types=[
            pltpu.VMEM((NF, BPW), jnp.float32),
            pltpu.VMEM((H, BPW), jnp.float32),
            pltpu.VMEM((2 * H,), jnp.float32),
            pltpu.VMEM((H,), jnp.float32),
            pltpu.VMEM((H, BPW), jnp.float32),
        ],
    )
    def compute(x_hbm, st_hbm, gb_hbm, cb_hbm, out_hbm,
                xv, sv, gbv, cbv, ov):
        wid = lax.axis_index("s") * NC + lax.axis_index("c")
        base = wid * BPW
        pltpu.sync_copy(x_hbm.at[:, pl.ds(base, BPW)], xv)
        pltpu.sync_copy(st_hbm.at[:, pl.ds(base, BPW)], sv)
        pltpu.sync_copy(gb_hbm, gbv)
        pltpu.sync_copy(cb_hbm, cbv)
        gbh = gbv[pl.ds(H, H)]
        cbh = cbv[...]

        def group(g, carry):
            gs = pl.ds(g * L, L)
            sh = [sv[h, gs] for h in range(H)]
            for k in range(H):
                acc_u = xv[2 * H * H + k, gs] + gbh[k]
                acc_c = xv[2 * H * H + H + k, gs] + cbh[k]
                for h in range(H):
                    acc_u = acc_u + sh[h] * xv[h * H + k, gs]
                    acc_c = acc_c + sh[h] * xv[H * H + h * H + k, gs]
                u_gate = _sigmoid(acc_u)
                c = _tanh(acc_c)
                ov[k, gs] = u_gate * sh[k] + (1.0 - u_gate) * c
            return carry

        lax.fori_loop(0, BPW // L, group, 0)
        pltpu.sync_copy(ov, out_hbm.at[:, pl.ds(base, BPW)])

    x = gather_rows(u_idx, i_idx, GT, CT, GIT, CIT)
    out = compute(x, ST, gate_bias, candidate_bias)
    return out.T


# speedup vs baseline: 9.8652x; 1.2002x over previous
import functools

import jax
import jax.numpy as jnp
from jax import lax
from jax.experimental import pallas as pl
from jax.experimental.pallas import tpu as pltpu
from jax.experimental.pallas import tpu_sc as plsc

NC = 2
NS = 16
NW = NC * NS
L = 16


def _sigmoid(x):
    e = jnp.exp(-jnp.abs(x))
    return jnp.where(x >= 0, 1.0 / (1.0 + e), e / (1.0 + e))


def _tanh(x):
    e = jnp.exp(-2.0 * jnp.abs(x))
    t = (1.0 - e) / (1.0 + e)
    return jnp.where(x >= 0, t, -t)


@jax.jit
def kernel(inputs, state, gate_kernel_users, gate_kernel_items, gate_bias,
           candidate_kernel_users, candidate_kernel_items, candidate_bias):
    B, H = state.shape
    BPW = B // NW
    NGALL = B // L
    u_idx = inputs[:, 0].astype(jnp.int32)
    i_idx = inputs[:, 1].astype(jnp.int32)
    U1 = gate_kernel_users.shape[0]
    GT = gate_kernel_users.transpose(1, 2, 0).reshape(2 * H * H, U1)
    CT = candidate_kernel_users.transpose(1, 2, 0).reshape(H * H, U1)
    GIT = gate_kernel_items.T
    CIT = candidate_kernel_items.T
    ST = state.T
    T0 = (U1 // 3) // 128 * 128
    OFFS = (0, T0, 2 * T0)
    TW = (T0, T0, U1 - 2 * T0)
    TMAX = max(TW)
    NF = 2 * H * H + 2 * H
    NRING = 16

    mesh = plsc.VectorSubcoreMesh(
        core_axis_name="c", subcore_axis_name="s",
        num_cores=NC, num_subcores=NS)

    @functools.partial(
        pl.kernel,
        out_type=jax.ShapeDtypeStruct((NF, B), jnp.float32),
        mesh=mesh,
        scratch_types=[
            pltpu.VMEM((B,), jnp.int32),
            pltpu.VMEM((B,), jnp.int32),
            pltpu.VMEM((1, TMAX), jnp.float32),
            pltpu.VMEM((1, TMAX), jnp.float32),
            pltpu.VMEM((1, TMAX), jnp.float32),
            pltpu.VMEM((1, B + L), jnp.float32),
            pltpu.VMEM((1, B + L), jnp.float32),
            pltpu.VMEM((B + 3 * L,), jnp.int32),
            pltpu.VMEM((B + 3 * L,), jnp.int32),
            pltpu.SemaphoreType.DMA,
            pltpu.SemaphoreType.DMA,
            pltpu.SemaphoreType.DMA,
            pltpu.SemaphoreType.DMA,
        ],
        compiler_params=pltpu.CompilerParams(needs_layout_passes=False),
    )
    def gather_rows(u_hbm, i_hbm, gt_hbm, ct_hbm, git_hbm, cit_hbm, x_hbm,
                    uv, iv, b0, b1, b2, oe, oo, lid, lpos, s0, s1, s2, so):
        wid = lax.axis_index("s") * NC + lax.axis_index("c")
        pltpu.sync_copy(u_hbm, uv)
        pltpu.sync_copy(i_hbm, iv)
        zero16 = jnp.zeros((L,), jnp.int32)
        iota16 = lax.iota(jnp.int32, L)
        bufs = (b0, b1, b2)
        sems = (s0, s1, s2)
        orows = (oe, oo)

        parts = []
        off = jnp.int32(0)
        for t in range(3):
            lo, hi = OFFS[t], OFFS[t] + TW[t]
            start = off

            def build(g, o, lo=lo, hi=hi):
                u = uv[pl.ds(g * L, L)]
                m = (u >= lo) & (u < hi)
                plsc.store_compressed(lid.at[pl.ds(o, L)], u - lo, mask=m)
                plsc.store_compressed(lpos.at[pl.ds(o, L)],
                                      g * L + iota16, mask=m)
                return o + plsc.all_reduce_population_count(m)[0]

            off = lax.fori_loop(0, NGALL, build, off)
            cnt = off - start
            lid[pl.ds(off, L)] = jnp.zeros((L,), jnp.int32)
            lpos[pl.ds(off, L)] = jnp.full((L,), B, jnp.int32)
            off = off + L
            parts.append((start, (cnt + L - 1) // L))

        def seg_copy(src_hbm, src_row, t, bi):
            return pltpu.make_async_copy(
                src_hbm.at[pl.ds(src_row, 1), pl.ds(OFFS[t], TW[t])],
                bufs[bi].at[:, pl.ds(0, TW[t])], sems[bi])

        def seg_gather(ids_v, t, bi, orow):
            lo, hi = OFFS[t], OFFS[t] + TW[t]

            def body(g, carry):
                u = ids_v[pl.ds(g * L, L)]
                m = (u >= lo) & (u < hi)
                v = plsc.load_gather(bufs[bi], [zero16, u - lo], mask=m)
                if t == 0:
                    orow[0, pl.ds(g * L, L)] = v
                else:
                    plsc.store_scatter(orow, [zero16, g * L + iota16], v,
                                       mask=m)
                return carry

            lax.fori_loop(0, NGALL, body, 0)

        def seg_gather_packed(t, bi, orow):
            start, ng = parts[t]

            def body(g, carry):
                o = start + g * L
                ul = lid[pl.ds(o, L)]
                pos = lpos[pl.ds(o, L)]
                v = plsc.load_gather(bufs[bi], [zero16, ul])
                plsc.store_scatter(orow, [zero16, pos], v)
                return carry

            lax.fori_loop(0, ng, body, 0)

        def out_copy(orow, out_row):
            return pltpu.make_async_copy(
                orow.at[:, pl.ds(0, B)], x_hbm.at[pl.ds(out_row, 1), :], so)

        rows = []
        for j in range(8):
            r = wid * 8 + j
            h = r // H
            k = lax.rem(r, H)
            rows.append((gt_hbm, h * 2 * H + H + k, r))
        for j in range(8):
            r = wid * 8 + j
            rows.append((ct_hbm, r, H * H + r))

        segs = [(ri, t) for ri in range(NRING) for t in range(3)]
        for i in range(3):
            ri, t = segs[i]
            seg_copy(rows[ri][0], rows[ri][1], t, i).start()
        for i, (ri, t) in enumerate(segs):
            bi = i % 3
            src_hbm, src_row, out_row = rows[ri]
            seg_copy(src_hbm, src_row, t, bi).wait()
            if t == 0 and ri >= 2:
                out_copy(orows[ri % 2], rows[ri - 2][2]).wait()
            seg_gather_packed(t, bi, orows[ri % 2])
            if i + 3 < len(segs):
                nri, nt = segs[i + 3]
                seg_copy(rows[nri][0], rows[nri][1], nt, bi).start()
            if t == 2:
                out_copy(orows[ri % 2], out_row).start()
        out_copy(orows[0], rows[NRING - 2][2]).wait()
        out_copy(orows[1], rows[NRING - 1][2]).wait()

        @pl.when(wid < NS)
        def _():
            for t in range(3):
                seg_copy(git_hbm, H + wid, t, t).start()
            for t in range(3):
                seg_copy(git_hbm, H + wid, t, t).wait()
                seg_gather(iv, t, t, oe)
            pltpu.sync_copy(oe.at[:, pl.ds(0, B)],
                            x_hbm.at[pl.ds(2 * H * H + wid, 1), :])

        @pl.when(wid >= NS)
        def _():
            for t in range(3):
                seg_copy(cit_hbm, wid - NS, t, t).start()
            for t in range(3):
                seg_copy(cit_hbm, wid - NS, t, t).wait()
                seg_gather(iv, t, t, oe)
            pltpu.sync_copy(
                oe.at[:, pl.ds(0, B)],
                x_hbm.at[pl.ds(2 * H * H + H + wid - NS, 1), :])

    @functools.partial(
        pl.kernel,
        out_type=jax.ShapeDtypeStruct((H, B), jnp.float32),
        mesh=mesh,
        scratch_types=[
            pltpu.VMEM((NF, BPW), jnp.float32),
            pltpu.VMEM((H, BPW), jnp.float32),
            pltpu.VMEM((2 * H,), jnp.float32),
            pltpu.VMEM((H,), jnp.float32),
            pltpu.VMEM((H, BPW), jnp.float32),
        ],
    )
    def compute(x_hbm, st_hbm, gb_hbm, cb_hbm, out_hbm,
                xv, sv, gbv, cbv, ov):
        wid = lax.axis_index("s") * NC + lax.axis_index("c")
        base = wid * BPW
        pltpu.sync_copy(x_hbm.at[:, pl.ds(base, BPW)], xv)
        pltpu.sync_copy(st_hbm.at[:, pl.ds(base, BPW)], sv)
        pltpu.sync_copy(gb_hbm, gbv)
        pltpu.sync_copy(cb_hbm, cbv)
        gbh = gbv[pl.ds(H, H)]
        cbh = cbv[...]

        def group(g, carry):
            gs = pl.ds(g * L, L)
            sh = [sv[h, gs] for h in range(H)]
            for k in range(H):
                acc_u = xv[2 * H * H + k, gs] + gbh[k]
                acc_c = xv[2 * H * H + H + k, gs] + cbh[k]
                for h in range(H):
                    acc_u = acc_u + sh[h] * xv[h * H + k, gs]
                    acc_c = acc_c + sh[h] * xv[H * H + h * H + k, gs]
                u_gate = _sigmoid(acc_u)
                c = _tanh(acc_c)
                ov[k, gs] = u_gate * sh[k] + (1.0 - u_gate) * c
            return carry

        lax.fori_loop(0, BPW // L, group, 0)
        pltpu.sync_copy(ov, out_hbm.at[:, pl.ds(base, BPW)])

    x = gather_rows(u_idx, i_idx, GT, CT, GIT, CIT)
    out = compute(x, ST, gate_bias, candidate_bias)
    return out.T
